# baseline (speedup 1.0000x reference)
import functools

import jax
import jax.numpy as jnp
from jax import lax
from jax.experimental import pallas as pl
from jax.experimental.pallas import tpu as pltpu
from jax.experimental.pallas import tpu_sc as plsc
from jax.experimental.layout import Format, Layout, with_layout_constraint

_NUM_CORES = 2
_NUM_SUBCORES = 16
_NUM_WORKERS = _NUM_CORES * _NUM_SUBCORES
_CHUNK = 800


def _sc_gather(weight, idx, num_indices, dim):
    b_per_w = num_indices // _NUM_WORKERS
    steps = b_per_w // _CHUNK
    mesh = plsc.VectorSubcoreMesh(core_axis_name="c", subcore_axis_name="s")

    @functools.partial(
        pl.kernel,
        mesh=mesh,
        compiler_params=pltpu.CompilerParams(use_tc_tiling_on_sc=False),
        out_type=jax.ShapeDtypeStruct((num_indices, dim), jnp.float32),
        scratch_types=[
            pltpu.VMEM((2, _CHUNK), jnp.int32),
            pltpu.VMEM((2, _CHUNK, dim), jnp.float32),
            pltpu.SemaphoreType.DMA((2,)),
            pltpu.SemaphoreType.DMA((2,)),
            pltpu.SemaphoreType.DMA((2,)),
        ],
    )
    def k(table_hbm, idx_hbm, out_hbm, idx_v, rows_v, sem_i, sem_g, sem_o):
        wid = lax.axis_index("s") * _NUM_CORES + lax.axis_index("c")
        base = wid * b_per_w

        def idx_copy(step, b):
            return pltpu.make_async_copy(
                idx_hbm.at[pl.ds(base + step * _CHUNK, _CHUNK)],
                idx_v.at[b],
                sem_i.at[b],
            )

        def gather_copy(b):
            return pltpu.make_async_copy(
                table_hbm.at[idx_v.at[b]], rows_v.at[b], sem_g.at[b]
            )

        def out_copy(step, b):
            return pltpu.make_async_copy(
                rows_v.at[b],
                out_hbm.at[pl.ds(base + step * _CHUNK, _CHUNK)],
                sem_o.at[b],
            )

        idx_copy(0, 0).start()
        idx_copy(1, 1).start()
        idx_copy(0, 0).wait()
        gather_copy(0).start()

        @pl.loop(0, steps, step=2)
        def _(i):
            for b in range(2):
                step = i + b
                b1 = 1 - b

                @pl.when(step + 1 < steps)
                def _():
                    idx_copy(step + 1, b1).wait()

                    @pl.when(step >= 1)
                    def _():
                        out_copy(step - 1, b1).wait()

                    gather_copy(b1).start()

                gather_copy(b).wait()
                out_copy(step, b).start()

                @pl.when(step + 2 < steps)
                def _():
                    idx_copy(step + 2, b).start()

        out_copy(steps - 1, (steps - 1) % 2).wait()

    return k(weight, idx)


def kernel(mask, weight):
    batch, hist = mask.shape
    _, dim = weight.shape
    num_indices = batch * hist
    idx = mask.reshape(num_indices)
    weight = with_layout_constraint(
        weight, Layout(major_to_minor=(0, 1), tiling=((16,),))
    )
    out = _sc_gather(weight, idx, num_indices, dim)
    return out.reshape(batch, hist, dim)


# speedup vs baseline: 1.0021x; 1.0021x over previous
import functools

import jax
import jax.numpy as jnp
from jax import lax
from jax.experimental import pallas as pl
from jax.experimental.pallas import tpu as pltpu
from jax.experimental.pallas import tpu_sc as plsc
from jax.experimental.layout import Format, Layout, with_layout_constraint

_NUM_CORES = 2
_NUM_SUBCORES = 16
_NUM_WORKERS = _NUM_CORES * _NUM_SUBCORES
_CHUNK = 800


def _sc_gather(weight, idx, num_indices, dim):
    b_per_w = num_indices // _NUM_WORKERS
    steps = b_per_w // _CHUNK
    mesh = plsc.VectorSubcoreMesh(core_axis_name="c", subcore_axis_name="s")

    @functools.partial(
        pl.kernel,
        mesh=mesh,
        compiler_params=pltpu.CompilerParams(use_tc_tiling_on_sc=False),
        out_type=jax.ShapeDtypeStruct((num_indices, dim), jnp.float32),
        scratch_types=[
            pltpu.VMEM((2, _CHUNK), jnp.int32),
            pltpu.VMEM((2, _CHUNK, dim), jnp.float32),
            pltpu.SemaphoreType.DMA((2,)),
            pltpu.SemaphoreType.DMA((2,)),
            pltpu.SemaphoreType.DMA((2,)),
        ],
    )
    def k(table_hbm, idx_hbm, out_hbm, idx_v, rows_v, sem_i, sem_g, sem_o):
        wid = lax.axis_index("s") * _NUM_CORES + lax.axis_index("c")
        base = wid * b_per_w

        def idx_copy(step, b):
            return pltpu.make_async_copy(
                idx_hbm.at[pl.ds(base + step * _CHUNK, _CHUNK)],
                idx_v.at[b],
                sem_i.at[b],
            )

        def gather_copy(b):
            return pltpu.make_async_copy(
                table_hbm.at[idx_v.at[b]], rows_v.at[b], sem_g.at[b]
            )

        def out_copy(step, b):
            return pltpu.make_async_copy(
                rows_v.at[b],
                out_hbm.at[pl.ds(base + step * _CHUNK, _CHUNK)],
                sem_o.at[b],
            )

        idx_copy(0, 0).start()
        idx_copy(1, 1).start()
        idx_copy(0, 0).wait()
        gather_copy(0).start()

        @pl.loop(0, steps, step=2)
        def _(i):
            for b in range(2):
                step = i + b
                b1 = 1 - b

                @pl.when(step + 1 < steps)
                def _():
                    idx_copy(step + 1, b1).wait()

                    @pl.when(step >= 1)
                    def _():
                        out_copy(step - 1, b1).wait()

                    gather_copy(b1).start()

                gather_copy(b).wait()
                out_copy(step, b).start()

                @pl.when(step + 2 < steps)
                def _():
                    idx_copy(step + 2, b).start()

        out_copy(steps - 1, (steps - 1) % 2).wait()

    return k(weight, idx)


def kernel(mask, weight):
    batch, hist = mask.shape
    _, dim = weight.shape
    num_indices = batch * hist
    idx = mask.reshape(num_indices) * 2
    weight = with_layout_constraint(
        weight, Layout(major_to_minor=(0, 1), tiling=((16,),))
    )
    out = _sc_gather(weight, idx, num_indices, dim)
    return out.reshape(batch, hist, dim)
